# baseline (speedup 1.0000x reference)
import functools

import jax
import jax.numpy as jnp
from jax import lax
from jax.experimental import pallas as pl
from jax.experimental.pallas import tpu as pltpu
from jax.experimental.pallas import tpu_sc as plsc

NC = 2
NS = 16
NW = NC * NS
CH = 128
PIECES = 2


def _sc_gather_packed(table_p, sidx_e, sidx_o, didx_e, didx_o):
    n_chunks = sidx_e.shape[0]
    _, Dw = table_p.shape
    H = n_chunks * CH // 2
    hc = CH // 2
    mesh = plsc.VectorSubcoreMesh(core_axis_name="c", subcore_axis_name="s")

    @functools.partial(
        pl.kernel,
        out_type=(
            jax.ShapeDtypeStruct((H, 2 * Dw), jnp.int32),
            jax.ShapeDtypeStruct((H, 2 * Dw), jnp.int32),
        ),
        mesh=mesh,
        scratch_types=[
            pltpu.VMEM((1, hc), jnp.int32),
            pltpu.VMEM((hc, Dw), jnp.int32),
            pltpu.SemaphoreType.DMA,
        ],
        compiler_params=pltpu.CompilerParams(use_tc_tiling_on_sc=False),
    )
    def k(table_hbm, sidxe_hbm, sidxo_hbm, didxe_hbm, didxo_hbm,
          outs_hbm, outd_hbm, idx_v, rows_v, sem):
        wid = lax.axis_index("s") * NC + lax.axis_index("c")
        nloc = (n_chunks - wid + NW - 1) // NW

        def one(idx_hbm, out_hbm, c, col):
            pltpu.sync_copy(idx_hbm.at[c], idx_v)
            pltpu.async_copy(table_hbm.at[idx_v.at[0]], rows_v, sem).wait()
            pltpu.sync_copy(rows_v,
                            out_hbm.at[pl.ds(c * hc, hc), pl.ds(col, Dw)])

        @pl.loop(0, nloc)
        def _(j):
            c = wid + j * NW
            one(sidxe_hbm, outs_hbm, c, 0)
            one(sidxo_hbm, outs_hbm, c, Dw)
            one(didxe_hbm, outd_hbm, c, 0)
            one(didxo_hbm, outd_hbm, c, Dw)

    return k(table_p, sidx_e, sidx_o, didx_e, didx_o)


def _sc_scatter(ef2, didx_e3, didx_o3, zeros, n_nodes):
    n_chunks = didx_e3.shape[0]
    D = ef2.shape[1] // 2
    hc = CH // 2
    rpt = (n_nodes // NS) & ~7
    tail = n_nodes - rpt * NS
    mesh = plsc.VectorSubcoreMesh(core_axis_name="c", subcore_axis_name="s")

    @functools.partial(
        pl.kernel,
        out_type=jax.ShapeDtypeStruct((NC * n_nodes, D), jnp.float32),
        mesh=mesh,
        scratch_types=[
            pltpu.VMEM((1, hc), jnp.int32),
            pltpu.VMEM((1, hc), jnp.int32),
            pltpu.VMEM((hc, D), jnp.float32),
            pltpu.VMEM((hc, D), jnp.float32),
            pltpu.VMEM_SHARED((n_nodes, D), jnp.float32),
        ],
    )
    def k(ef_hbm, didxe_hbm, didxo_hbm, zeros_hbm, out_hbm,
          idxe_v, idxo_v, rows_e, rows_o, acc_s):
        cid = lax.axis_index("c")
        sid = lax.axis_index("s")
        wid = sid * NC + cid
        pltpu.sync_copy(zeros_hbm.at[pl.ds(sid * rpt, rpt)],
                        acc_s.at[pl.ds(sid * rpt, rpt)])

        @pl.when(jnp.logical_and(sid == NS - 1, tail > 0))
        def _():
            pltpu.sync_copy(zeros_hbm.at[pl.ds(NS * rpt, tail)],
                            acc_s.at[pl.ds(NS * rpt, tail)])

        plsc.subcore_barrier()
        nloc = (n_chunks - wid + NW - 1) // NW

        @pl.loop(0, nloc)
        def _(j):
            c = wid + j * NW
            pltpu.sync_copy(didxe_hbm.at[c], idxe_v)
            pltpu.sync_copy(didxo_hbm.at[c], idxo_v)
            pltpu.sync_copy(ef_hbm.at[pl.ds(c * hc, hc), pl.ds(0, D)], rows_e)
            pltpu.sync_copy(ef_hbm.at[pl.ds(c * hc, hc), pl.ds(D, D)], rows_o)
            pltpu.sync_copy(rows_e, acc_s.at[idxe_v.at[0]], add=True)
            pltpu.sync_copy(rows_o, acc_s.at[idxo_v.at[0]], add=True)

        plsc.subcore_barrier()
        pltpu.sync_copy(acc_s.at[pl.ds(sid * rpt, rpt)],
                        out_hbm.at[pl.ds(cid * n_nodes + sid * rpt, rpt)])

        @pl.when(jnp.logical_and(sid == NS - 1, tail > 0))
        def _():
            pltpu.sync_copy(acc_s.at[pl.ds(NS * rpt, tail)],
                            out_hbm.at[pl.ds(cid * n_nodes + NS * rpt, tail)])

    return k(ef2, didx_e3, didx_o3, zeros)


def _layer_norm(h, g, beta):
    mu = jnp.mean(h, axis=-1, keepdims=True)
    var = jnp.mean((h - mu) * (h - mu), axis=-1, keepdims=True)
    return (h - mu) * lax.rsqrt(var + 1e-5) * g + beta


def _tc_edge_mlp(srcp, dstp, ef2, A_lo, A_hi, B_lo, B_hi, W0e2, b02,
                 W12, b12, W22, b22, g, beta):
    H, D2 = ef2.shape
    D = D2 // 2
    BH = 1000
    grid = (H // BH,)
    bf16 = jnp.bfloat16

    def body(src_r, dst_r, ef_r, alo_r, ahi_r, blo_r, bhi_r, w0e_r, b0_r,
             w1_r, b1_r, w2_r, b2_r, g_r, beta_r, out_r):
        dot = functools.partial(jnp.dot, preferred_element_type=jnp.float32)

        def unpack(xi):
            lo = lax.bitcast_convert_type(xi << 16, jnp.float32)
            hi = lax.bitcast_convert_type(xi & jnp.int32(-65536), jnp.float32)
            return lo.astype(bf16), hi.astype(bf16)

        lo_s, hi_s = unpack(src_r[...])
        lo_d, hi_d = unpack(dst_r[...])
        ef = ef_r[...]
        x = (dot(lo_s, alo_r[...]) + dot(hi_s, ahi_r[...])
             + dot(lo_d, blo_r[...]) + dot(hi_d, bhi_r[...])
             + dot(ef.astype(bf16), w0e_r[...]) + b0_r[...])
        h = jnp.maximum(x, 0.0)
        h = jnp.maximum(dot(h.astype(bf16), w1_r[...]) + b1_r[...], 0.0)
        h = dot(h.astype(bf16), w2_r[...]) + b2_r[...]
        hL = lax.slice_in_dim(h, 0, D, axis=1)
        hR = lax.slice_in_dim(h, D, 2 * D, axis=1)
        ln = jnp.concatenate(
            [_layer_norm(hL, g_r[...], beta_r[...]),
             _layer_norm(hR, g_r[...], beta_r[...])], axis=1)
        out_r[...] = ef + ln

    blk = lambda i: (i, 0)
    full = lambda i: (0, 0)
    return pl.pallas_call(
        body,
        grid=grid,
        in_specs=[
            pl.BlockSpec((BH, D), blk),
            pl.BlockSpec((BH, D), blk),
            pl.BlockSpec((BH, D2), blk),
            pl.BlockSpec((D, D2), full),
            pl.BlockSpec((D, D2), full),
            pl.BlockSpec((D, D2), full),
            pl.BlockSpec((D, D2), full),
            pl.BlockSpec((D2, D2), full),
            pl.BlockSpec((1, D2), full),
            pl.BlockSpec((D2, D2), full),
            pl.BlockSpec((1, D2), full),
            pl.BlockSpec((D2, D2), full),
            pl.BlockSpec((1, D2), full),
            pl.BlockSpec((1, D), full),
            pl.BlockSpec((1, D), full),
        ],
        out_specs=pl.BlockSpec((BH, D2), blk),
        out_shape=jax.ShapeDtypeStruct((H, D2), jnp.float32),
    )(srcp, dstp, ef2, A_lo, A_hi, B_lo, B_hi, W0e2, b02, W12, b12,
      W22, b22, g, beta)


def _tc_node_mlp(nf, parts_list, W0n, W0a, b0, W1, b1, W2, b2, g, beta):
    N, D = nf.shape
    BN = 1000
    grid = (N // BN,)
    n_parts = 2 * len(parts_list)

    def body(*refs):
        nf_r = refs[0]
        part_rs = refs[1:1 + n_parts]
        (w0n_r, w0a_r, b0_r, w1_r, b1_r, w2_r, b2_r, g_r, beta_r,
         out_r) = refs[1 + n_parts:]
        dot = functools.partial(jnp.dot, preferred_element_type=jnp.float32)
        agg = part_rs[0][...]
        for pr in part_rs[1:]:
            agg = agg + pr[...]
        x = dot(nf_r[...], w0n_r[...]) + dot(agg, w0a_r[...]) + b0_r[...]
        h = jnp.maximum(x, 0.0)
        h = jnp.maximum(dot(h, w1_r[...]) + b1_r[...], 0.0)
        h = dot(h, w2_r[...]) + b2_r[...]
        out_r[...] = nf_r[...] + _layer_norm(h, g_r[...], beta_r[...])

    blk = lambda i: (i, 0)
    full = lambda i: (0, 0)
    flat_parts = []
    for parts in parts_list:
        flat_parts.append(lax.slice_in_dim(parts, 0, N, axis=0))
        flat_parts.append(lax.slice_in_dim(parts, N, 2 * N, axis=0))
    return pl.pallas_call(
        body,
        grid=grid,
        in_specs=(
            [pl.BlockSpec((BN, D), blk)] * (1 + n_parts)
            + [
                pl.BlockSpec((D, D), full),
                pl.BlockSpec((D, D), full),
                pl.BlockSpec((1, D), full),
                pl.BlockSpec((D, D), full),
                pl.BlockSpec((1, D), full),
                pl.BlockSpec((D, D), full),
                pl.BlockSpec((1, D), full),
                pl.BlockSpec((1, D), full),
                pl.BlockSpec((1, D), full),
            ]
        ),
        out_specs=pl.BlockSpec((BN, D), blk),
        out_shape=jax.ShapeDtypeStruct((N, D), jnp.float32),
    )(nf, *flat_parts, W0n, W0a, b0, W1, b1, W2, b2, g, beta)


def kernel(node_features, edge_features, edge_index,
           edge_W0, edge_b0, edge_W1, edge_b1, edge_W2, edge_b2,
           edge_g, edge_beta,
           node_W0, node_b0, node_W1, node_b1, node_W2, node_b2,
           node_g, node_beta):
    N, DN = node_features.shape
    E, DE = edge_features.shape
    P = edge_W0.shape[0]
    n_chunks = E // CH
    cpp = n_chunks // PIECES
    Ep = cpp * CH
    bf16 = jnp.bfloat16
    i2 = jnp.eye(2, dtype=jnp.float32)

    hc = CH // 2
    def eo_split(idx):
        eo = idx.reshape(n_chunks, hc, 2)
        return (eo[:, :, 0].reshape(n_chunks, 1, hc),
                eo[:, :, 1].reshape(n_chunks, 1, hc))

    sidx_e3, sidx_o3 = eo_split(edge_index[0])
    didx_e3, didx_o3 = eo_split(edge_index[1])
    pieceof = lambda a: [a[k * cpp:(k + 1) * cpp] for k in range(PIECES)]
    sidxe_p, sidxo_p = pieceof(sidx_e3), pieceof(sidx_o3)
    didxe_p, didxo_p = pieceof(didx_e3), pieceof(didx_o3)
    zeros = jnp.zeros((N, DE), dtype=jnp.float32)
    row = lambda b: b.reshape(1, -1)
    kron2 = lambda w: jnp.kron(i2, w).astype(bf16)
    tile2 = lambda b: jnp.concatenate([b, b]).reshape(1, -1)

    def pack_cols(x16):
        h = x16.shape[1] // 2
        u = lax.bitcast_convert_type(x16, jnp.uint16).astype(jnp.uint32)
        packed = u[:, :h] | (u[:, h:] << 16)
        return lax.bitcast_convert_type(packed, jnp.int32)

    nf = node_features
    ef_p = [lax.slice_in_dim(edge_features, k * Ep, (k + 1) * Ep, axis=0)
            .reshape(Ep // 2, 2 * DE) for k in range(PIECES)]
    for i in range(P):
        W0s, W0d, W0e = (edge_W0[i, :DN], edge_W0[i, DN:2 * DN],
                         edge_W0[i, 2 * DN:])
        ew = (kron2(W0s[:DN // 2]), kron2(W0s[DN // 2:]),
              kron2(W0d[:DN // 2]), kron2(W0d[DN // 2:]),
              kron2(W0e), tile2(edge_b0[i]),
              kron2(edge_W1[i]), tile2(edge_b1[i]),
              kron2(edge_W2[i]), tile2(edge_b2[i]),
              row(edge_g[i]), row(edge_beta[i]))
        table_p = pack_cols(nf.astype(bf16))
        parts_list = []
        new_ef_p = []
        for k in range(PIECES):
            srcp, dstp = _sc_gather_packed(table_p, sidxe_p[k], sidxo_p[k],
                                           didxe_p[k], didxo_p[k])
            efk = _tc_edge_mlp(srcp, dstp, ef_p[k], *ew)
            new_ef_p.append(efk)
            parts_list.append(
                _sc_scatter(efk, didxe_p[k], didxo_p[k], zeros, N))
        ef_p = new_ef_p
        nf = _tc_node_mlp(
            nf, parts_list,
            node_W0[i, :DN], node_W0[i, DN:],
            row(node_b0[i]), node_W1[i], row(node_b1[i]),
            node_W2[i], row(node_b2[i]), row(node_g[i]), row(node_beta[i]))
    return nf


# speedup vs baseline: 1.6426x; 1.6426x over previous
import functools

import jax
import jax.numpy as jnp
from jax import lax
from jax.experimental import pallas as pl
from jax.experimental.pallas import tpu as pltpu
from jax.experimental.pallas import tpu_sc as plsc

NC = 2
NS = 16
NW = NC * NS
CH = 128
PIECES = 2


def _sc_gather(table, sidx3d, didx3d):
    n_chunks = sidx3d.shape[0]
    _, D = table.shape
    Ep = n_chunks * CH
    mesh = plsc.VectorSubcoreMesh(core_axis_name="c", subcore_axis_name="s")

    @functools.partial(
        pl.kernel,
        out_type=(
            jax.ShapeDtypeStruct((Ep, D), jnp.float32),
            jax.ShapeDtypeStruct((Ep, D), jnp.float32),
        ),
        mesh=mesh,
        scratch_types=[
            pltpu.VMEM((1, CH), jnp.int32),
            pltpu.VMEM((1, CH), jnp.int32),
            pltpu.VMEM((1, CH), jnp.int32),
            pltpu.VMEM((1, CH), jnp.int32),
            pltpu.VMEM((CH, D), jnp.float32),
            pltpu.VMEM((CH, D), jnp.float32),
            pltpu.SemaphoreType.DMA,
            pltpu.SemaphoreType.DMA,
            pltpu.SemaphoreType.DMA,
        ],
    )
    def k(table_hbm, sidx_hbm, didx_hbm, outs_hbm, outd_hbm,
          sidx_a, didx_a, sidx_b, didx_b, rows_a, rows_b,
          sem_g, sem_wa, sem_wb):
        wid = lax.axis_index("s") * NC + lax.axis_index("c")
        nloc = (n_chunks - wid + NW - 1) // NW

        def chunk_of(j):
            return wid + j * NW

        def prefetch(j, si, di):
            @pl.when(j < nloc)
            def _():
                pltpu.sync_copy(sidx_hbm.at[chunk_of(j)], si)
                pltpu.sync_copy(didx_hbm.at[chunk_of(j)], di)

        def drain(rows, sem_w):
            pltpu.make_async_copy(table_hbm.at[pl.ds(0, CH)], rows,
                                  sem_w).wait()

        def unit(idx_v, rows, sem_w, out_hbm, c, do_drain):
            @pl.when(do_drain)
            def _():
                drain(rows, sem_w)

            pltpu.async_copy(table_hbm.at[idx_v.at[0]], rows, sem_g).wait()
            pltpu.async_copy(rows, out_hbm.at[pl.ds(c * CH, CH)], sem_w)

        prefetch(0, sidx_a, didx_a)

        @pl.loop(0, nloc)
        def _(j):
            c = chunk_of(j)
            even = j % 2 == 0

            @pl.when(even)
            def _():
                prefetch(j + 1, sidx_b, didx_b)
                unit(sidx_a, rows_a, sem_wa, outs_hbm, c, j > 0)
                unit(didx_a, rows_b, sem_wb, outd_hbm, c, j > 0)

            @pl.when(jnp.logical_not(even))
            def _():
                prefetch(j + 1, sidx_a, didx_a)
                unit(sidx_b, rows_a, sem_wa, outs_hbm, c, j > 0)
                unit(didx_b, rows_b, sem_wb, outd_hbm, c, j > 0)

        @pl.when(nloc > 0)
        def _():
            drain(rows_a, sem_wa)
            drain(rows_b, sem_wb)

    return k(table, sidx3d, didx3d)


def _sc_scatter(ef_piece, didx3d, zeros, n_nodes):
    n_chunks = didx3d.shape[0]
    D = ef_piece.shape[1]
    rpt = (n_nodes // NS) & ~7
    tail = n_nodes - rpt * NS
    mesh = plsc.VectorSubcoreMesh(core_axis_name="c", subcore_axis_name="s")

    @functools.partial(
        pl.kernel,
        out_type=jax.ShapeDtypeStruct((NC * n_nodes, D), jnp.float32),
        mesh=mesh,
        scratch_types=[
            pltpu.VMEM((1, CH), jnp.int32),
            pltpu.VMEM((CH, D), jnp.float32),
            pltpu.VMEM_SHARED((n_nodes, D), jnp.float32),
        ],
    )
    def k(ef_hbm, didx_hbm, zeros_hbm, out_hbm, idx_v, rows_v, acc_s):
        cid = lax.axis_index("c")
        sid = lax.axis_index("s")
        wid = sid * NC + cid
        pltpu.sync_copy(zeros_hbm.at[pl.ds(sid * rpt, rpt)],
                        acc_s.at[pl.ds(sid * rpt, rpt)])

        @pl.when(jnp.logical_and(sid == NS - 1, tail > 0))
        def _():
            pltpu.sync_copy(zeros_hbm.at[pl.ds(NS * rpt, tail)],
                            acc_s.at[pl.ds(NS * rpt, tail)])

        plsc.subcore_barrier()
        nloc = (n_chunks - wid + NW - 1) // NW

        @pl.loop(0, nloc)
        def _(j):
            c = wid + j * NW
            pltpu.sync_copy(didx_hbm.at[c], idx_v)
            pltpu.sync_copy(ef_hbm.at[pl.ds(c * CH, CH)], rows_v)
            pltpu.sync_copy(rows_v, acc_s.at[idx_v.at[0]], add=True)

        plsc.subcore_barrier()
        pltpu.sync_copy(acc_s.at[pl.ds(sid * rpt, rpt)],
                        out_hbm.at[pl.ds(cid * n_nodes + sid * rpt, rpt)])

        @pl.when(jnp.logical_and(sid == NS - 1, tail > 0))
        def _():
            pltpu.sync_copy(acc_s.at[pl.ds(NS * rpt, tail)],
                            out_hbm.at[pl.ds(cid * n_nodes + NS * rpt, tail)])

    return k(ef_piece, didx3d, zeros)


def _layer_norm(h, g, beta):
    mu = jnp.mean(h, axis=-1, keepdims=True)
    var = jnp.mean((h - mu) * (h - mu), axis=-1, keepdims=True)
    return (h - mu) * lax.rsqrt(var + 1e-5) * g + beta


def _tc_edge_mlp(src, dst, ef, W0s, W0d, W0e, b0, W1, b1, W2, b2, g, beta):
    Ep, D = ef.shape
    BE = 2000
    grid = (Ep // BE,)

    def body(src_r, dst_r, ef_r, w0s_r, w0d_r, w0e_r, b0_r, w1_r, b1_r,
             w2_r, b2_r, g_r, beta_r, out_r):
        dot = functools.partial(jnp.dot, preferred_element_type=jnp.float32)
        x = (dot(src_r[...], w0s_r[...]) + dot(dst_r[...], w0d_r[...])
             + dot(ef_r[...], w0e_r[...]) + b0_r[...])
        h = jnp.maximum(x, 0.0)
        h = jnp.maximum(dot(h, w1_r[...]) + b1_r[...], 0.0)
        h = dot(h, w2_r[...]) + b2_r[...]
        out_r[...] = ef_r[...] + _layer_norm(h, g_r[...], beta_r[...])

    blk = lambda i: (i, 0)
    full = lambda i: (0, 0)
    return pl.pallas_call(
        body,
        grid=grid,
        in_specs=[
            pl.BlockSpec((BE, D), blk),
            pl.BlockSpec((BE, D), blk),
            pl.BlockSpec((BE, D), blk),
            pl.BlockSpec((D, D), full),
            pl.BlockSpec((D, D), full),
            pl.BlockSpec((D, D), full),
            pl.BlockSpec((1, D), full),
            pl.BlockSpec((D, D), full),
            pl.BlockSpec((1, D), full),
            pl.BlockSpec((D, D), full),
            pl.BlockSpec((1, D), full),
            pl.BlockSpec((1, D), full),
            pl.BlockSpec((1, D), full),
        ],
        out_specs=pl.BlockSpec((BE, D), blk),
        out_shape=jax.ShapeDtypeStruct((Ep, D), jnp.float32),
    )(src, dst, ef, W0s, W0d, W0e, b0, W1, b1, W2, b2, g, beta)


def _tc_node_mlp(nf, parts_list, W0n, W0a, b0, W1, b1, W2, b2, g, beta):
    N, D = nf.shape
    BN = 1000
    grid = (N // BN,)
    n_parts = 2 * len(parts_list)

    def body(*refs):
        nf_r = refs[0]
        part_rs = refs[1:1 + n_parts]
        (w0n_r, w0a_r, b0_r, w1_r, b1_r, w2_r, b2_r, g_r, beta_r,
         out_r) = refs[1 + n_parts:]
        dot = functools.partial(jnp.dot, preferred_element_type=jnp.float32)
        agg = part_rs[0][...]
        for pr in part_rs[1:]:
            agg = agg + pr[...]
        x = dot(nf_r[...], w0n_r[...]) + dot(agg, w0a_r[...]) + b0_r[...]
        h = jnp.maximum(x, 0.0)
        h = jnp.maximum(dot(h, w1_r[...]) + b1_r[...], 0.0)
        h = dot(h, w2_r[...]) + b2_r[...]
        out_r[...] = nf_r[...] + _layer_norm(h, g_r[...], beta_r[...])

    blk = lambda i: (i, 0)
    full = lambda i: (0, 0)
    flat_parts = []
    for parts in parts_list:
        flat_parts.append(lax.slice_in_dim(parts, 0, N, axis=0))
        flat_parts.append(lax.slice_in_dim(parts, N, 2 * N, axis=0))
    return pl.pallas_call(
        body,
        grid=grid,
        in_specs=(
            [pl.BlockSpec((BN, D), blk)] * (1 + n_parts)
            + [
                pl.BlockSpec((D, D), full),
                pl.BlockSpec((D, D), full),
                pl.BlockSpec((1, D), full),
                pl.BlockSpec((D, D), full),
                pl.BlockSpec((1, D), full),
                pl.BlockSpec((D, D), full),
                pl.BlockSpec((1, D), full),
                pl.BlockSpec((1, D), full),
                pl.BlockSpec((1, D), full),
            ]
        ),
        out_specs=pl.BlockSpec((BN, D), blk),
        out_shape=jax.ShapeDtypeStruct((N, D), jnp.float32),
    )(nf, *flat_parts, W0n, W0a, b0, W1, b1, W2, b2, g, beta)


def kernel(node_features, edge_features, edge_index,
           edge_W0, edge_b0, edge_W1, edge_b1, edge_W2, edge_b2,
           edge_g, edge_beta,
           node_W0, node_b0, node_W1, node_b1, node_W2, node_b2,
           node_g, node_beta):
    N, DN = node_features.shape
    E, DE = edge_features.shape
    P = edge_W0.shape[0]
    n_chunks = E // CH
    cpp = n_chunks // PIECES
    sidx3d = edge_index[0].reshape(n_chunks, 1, CH)
    didx3d = edge_index[1].reshape(n_chunks, 1, CH)
    sidx_p = [sidx3d[k * cpp:(k + 1) * cpp] for k in range(PIECES)]
    didx_p = [didx3d[k * cpp:(k + 1) * cpp] for k in range(PIECES)]
    zeros = jnp.zeros((N, DE), dtype=jnp.float32)
    row = lambda b: b.reshape(1, -1)
    Ep = cpp * CH

    nf = node_features
    ef_p = [lax.slice_in_dim(edge_features, k * Ep, (k + 1) * Ep, axis=0)
            for k in range(PIECES)]
    for i in range(P):
        ew = (edge_W0[i, :DN], edge_W0[i, DN:2 * DN], edge_W0[i, 2 * DN:],
              row(edge_b0[i]), edge_W1[i], row(edge_b1[i]),
              edge_W2[i], row(edge_b2[i]), row(edge_g[i]), row(edge_beta[i]))
        parts_list = []
        new_ef_p = []
        for k in range(PIECES):
            src_rows, dst_rows = _sc_gather(nf, sidx_p[k], didx_p[k])
            efk = _tc_edge_mlp(src_rows, dst_rows, ef_p[k], *ew)
            new_ef_p.append(efk)
            parts_list.append(_sc_scatter(efk, didx_p[k], zeros, N))
        ef_p = new_ef_p
        nf = _tc_node_mlp(
            nf, parts_list,
            node_W0[i, :DN], node_W0[i, DN:],
            row(node_b0[i]), node_W1[i], row(node_b1[i]),
            node_W2[i], row(node_b2[i]), row(node_g[i]), row(node_beta[i]))
    return nf


# speedup vs baseline: 1.7332x; 1.0551x over previous
import functools

import jax
import jax.numpy as jnp
from jax import lax
from jax.experimental import pallas as pl
from jax.experimental.pallas import tpu as pltpu
from jax.experimental.pallas import tpu_sc as plsc

NC = 2
NS = 16
NW = NC * NS
CH = 128
PIECES = 2


def _sc_gather(table, sidx3d, didx3d):
    n_chunks = sidx3d.shape[0]
    _, D = table.shape
    Ep = n_chunks * CH
    mesh = plsc.VectorSubcoreMesh(core_axis_name="c", subcore_axis_name="s")

    @functools.partial(
        pl.kernel,
        out_type=(
            jax.ShapeDtypeStruct((Ep, D), jnp.float32),
            jax.ShapeDtypeStruct((Ep, D), jnp.float32),
        ),
        mesh=mesh,
        scratch_types=[
            pltpu.VMEM((1, CH), jnp.int32),
            pltpu.VMEM((1, CH), jnp.int32),
            pltpu.VMEM((1, CH), jnp.int32),
            pltpu.VMEM((1, CH), jnp.int32),
            pltpu.VMEM((CH, D), jnp.float32),
            pltpu.VMEM((CH, D), jnp.float32),
            pltpu.SemaphoreType.DMA,
            pltpu.SemaphoreType.DMA,
            pltpu.SemaphoreType.DMA,
        ],
    )
    def k(table_hbm, sidx_hbm, didx_hbm, outs_hbm, outd_hbm,
          sidx_a, didx_a, sidx_b, didx_b, rows_a, rows_b,
          sem_g, sem_wa, sem_wb):
        wid = lax.axis_index("s") * NC + lax.axis_index("c")
        nloc = (n_chunks - wid + NW - 1) // NW

        def chunk_of(j):
            return wid + j * NW

        def prefetch(j, si, di):
            @pl.when(j < nloc)
            def _():
                pltpu.sync_copy(sidx_hbm.at[chunk_of(j)], si)
                pltpu.sync_copy(didx_hbm.at[chunk_of(j)], di)

        def drain(rows, sem_w):
            pltpu.make_async_copy(table_hbm.at[pl.ds(0, CH)], rows,
                                  sem_w).wait()

        def unit(idx_v, rows, sem_w, out_hbm, c, do_drain):
            @pl.when(do_drain)
            def _():
                drain(rows, sem_w)

            pltpu.async_copy(table_hbm.at[idx_v.at[0]], rows, sem_g).wait()
            pltpu.async_copy(rows, out_hbm.at[pl.ds(c * CH, CH)], sem_w)

        prefetch(0, sidx_a, didx_a)

        @pl.loop(0, nloc)
        def _(j):
            c = chunk_of(j)
            even = j % 2 == 0

            @pl.when(even)
            def _():
                prefetch(j + 1, sidx_b, didx_b)
                unit(sidx_a, rows_a, sem_wa, outs_hbm, c, j > 0)
                unit(didx_a, rows_b, sem_wb, outd_hbm, c, j > 0)

            @pl.when(jnp.logical_not(even))
            def _():
                prefetch(j + 1, sidx_a, didx_a)
                unit(sidx_b, rows_a, sem_wa, outs_hbm, c, j > 0)
                unit(didx_b, rows_b, sem_wb, outd_hbm, c, j > 0)

        @pl.when(nloc > 0)
        def _():
            drain(rows_a, sem_wa)
            drain(rows_b, sem_wb)

    return k(table, sidx3d, didx3d)


def _sc_scatter(ef_piece, didx3d, zeros, n_nodes):
    n_chunks = didx3d.shape[0]
    D = ef_piece.shape[1]
    rpt = (n_nodes // NS) & ~7
    tail = n_nodes - rpt * NS
    mesh = plsc.VectorSubcoreMesh(core_axis_name="c", subcore_axis_name="s")

    @functools.partial(
        pl.kernel,
        out_type=jax.ShapeDtypeStruct((NC * n_nodes, D), jnp.float32),
        mesh=mesh,
        scratch_types=[
            pltpu.VMEM((1, CH), jnp.int32),
            pltpu.VMEM((1, CH), jnp.int32),
            pltpu.VMEM((CH, D), jnp.float32),
            pltpu.VMEM((CH, D), jnp.float32),
            pltpu.SemaphoreType.DMA,
            pltpu.SemaphoreType.DMA,
            pltpu.VMEM_SHARED((n_nodes, D), jnp.float32),
        ],
    )
    def k(ef_hbm, didx_hbm, zeros_hbm, out_hbm, idx_a, idx_b,
          rows_a, rows_b, sem_ra, sem_rb, acc_s):
        cid = lax.axis_index("c")
        sid = lax.axis_index("s")
        wid = sid * NC + cid
        pltpu.sync_copy(zeros_hbm.at[pl.ds(sid * rpt, rpt)],
                        acc_s.at[pl.ds(sid * rpt, rpt)])

        @pl.when(jnp.logical_and(sid == NS - 1, tail > 0))
        def _():
            pltpu.sync_copy(zeros_hbm.at[pl.ds(NS * rpt, tail)],
                            acc_s.at[pl.ds(NS * rpt, tail)])

        plsc.subcore_barrier()
        nloc = (n_chunks - wid + NW - 1) // NW

        def chunk_of(j):
            return wid + j * NW

        def prefetch(j, idx_v, rows, sem_r):
            @pl.when(j < nloc)
            def _():
                c = chunk_of(j)
                pltpu.sync_copy(didx_hbm.at[c], idx_v)
                pltpu.async_copy(ef_hbm.at[pl.ds(c * CH, CH)], rows, sem_r)

        def add_unit(idx_v, rows, sem_r):
            pltpu.make_async_copy(ef_hbm.at[pl.ds(0, CH)], rows,
                                  sem_r).wait()
            pltpu.sync_copy(rows, acc_s.at[idx_v.at[0]], add=True)

        prefetch(0, idx_a, rows_a, sem_ra)

        @pl.loop(0, nloc)
        def _(j):
            even = j % 2 == 0

            @pl.when(even)
            def _():
                prefetch(j + 1, idx_b, rows_b, sem_rb)
                add_unit(idx_a, rows_a, sem_ra)

            @pl.when(jnp.logical_not(even))
            def _():
                prefetch(j + 1, idx_a, rows_a, sem_ra)
                add_unit(idx_b, rows_b, sem_rb)

        plsc.subcore_barrier()
        pltpu.sync_copy(acc_s.at[pl.ds(sid * rpt, rpt)],
                        out_hbm.at[pl.ds(cid * n_nodes + sid * rpt, rpt)])

        @pl.when(jnp.logical_and(sid == NS - 1, tail > 0))
        def _():
            pltpu.sync_copy(acc_s.at[pl.ds(NS * rpt, tail)],
                            out_hbm.at[pl.ds(cid * n_nodes + NS * rpt, tail)])

    return k(ef_piece, didx3d, zeros)


def _layer_norm(h, g, beta):
    mu = jnp.mean(h, axis=-1, keepdims=True)
    var = jnp.mean((h - mu) * (h - mu), axis=-1, keepdims=True)
    return (h - mu) * lax.rsqrt(var + 1e-5) * g + beta


def _tc_edge_mlp(src, dst, ef, W0s, W0d, W0e, b0, W1, b1, W2, b2, g, beta):
    Ep, D = ef.shape
    BE = 2000
    grid = (Ep // BE,)

    def body(src_r, dst_r, ef_r, w0s_r, w0d_r, w0e_r, b0_r, w1_r, b1_r,
             w2_r, b2_r, g_r, beta_r, out_r):
        dot = functools.partial(jnp.dot, preferred_element_type=jnp.float32)
        x = (dot(src_r[...], w0s_r[...]) + dot(dst_r[...], w0d_r[...])
             + dot(ef_r[...], w0e_r[...]) + b0_r[...])
        h = jnp.maximum(x, 0.0)
        h = jnp.maximum(dot(h, w1_r[...]) + b1_r[...], 0.0)
        h = dot(h, w2_r[...]) + b2_r[...]
        out_r[...] = ef_r[...] + _layer_norm(h, g_r[...], beta_r[...])

    blk = lambda i: (i, 0)
    full = lambda i: (0, 0)
    return pl.pallas_call(
        body,
        grid=grid,
        in_specs=[
            pl.BlockSpec((BE, D), blk),
            pl.BlockSpec((BE, D), blk),
            pl.BlockSpec((BE, D), blk),
            pl.BlockSpec((D, D), full),
            pl.BlockSpec((D, D), full),
            pl.BlockSpec((D, D), full),
            pl.BlockSpec((1, D), full),
            pl.BlockSpec((D, D), full),
            pl.BlockSpec((1, D), full),
            pl.BlockSpec((D, D), full),
            pl.BlockSpec((1, D), full),
            pl.BlockSpec((1, D), full),
            pl.BlockSpec((1, D), full),
        ],
        out_specs=pl.BlockSpec((BE, D), blk),
        out_shape=jax.ShapeDtypeStruct((Ep, D), jnp.float32),
    )(src, dst, ef, W0s, W0d, W0e, b0, W1, b1, W2, b2, g, beta)


def _tc_node_mlp(nf, parts_list, W0n, W0a, b0, W1, b1, W2, b2, g, beta):
    N, D = nf.shape
    BN = 1000
    grid = (N // BN,)
    n_parts = 2 * len(parts_list)

    def body(*refs):
        nf_r = refs[0]
        part_rs = refs[1:1 + n_parts]
        (w0n_r, w0a_r, b0_r, w1_r, b1_r, w2_r, b2_r, g_r, beta_r,
         out_r) = refs[1 + n_parts:]
        dot = functools.partial(jnp.dot, preferred_element_type=jnp.float32)
        agg = part_rs[0][...]
        for pr in part_rs[1:]:
            agg = agg + pr[...]
        x = dot(nf_r[...], w0n_r[...]) + dot(agg, w0a_r[...]) + b0_r[...]
        h = jnp.maximum(x, 0.0)
        h = jnp.maximum(dot(h, w1_r[...]) + b1_r[...], 0.0)
        h = dot(h, w2_r[...]) + b2_r[...]
        out_r[...] = nf_r[...] + _layer_norm(h, g_r[...], beta_r[...])

    blk = lambda i: (i, 0)
    full = lambda i: (0, 0)
    flat_parts = []
    for parts in parts_list:
        flat_parts.append(lax.slice_in_dim(parts, 0, N, axis=0))
        flat_parts.append(lax.slice_in_dim(parts, N, 2 * N, axis=0))
    return pl.pallas_call(
        body,
        grid=grid,
        in_specs=(
            [pl.BlockSpec((BN, D), blk)] * (1 + n_parts)
            + [
                pl.BlockSpec((D, D), full),
                pl.BlockSpec((D, D), full),
                pl.BlockSpec((1, D), full),
                pl.BlockSpec((D, D), full),
                pl.BlockSpec((1, D), full),
                pl.BlockSpec((D, D), full),
                pl.BlockSpec((1, D), full),
                pl.BlockSpec((1, D), full),
                pl.BlockSpec((1, D), full),
            ]
        ),
        out_specs=pl.BlockSpec((BN, D), blk),
        out_shape=jax.ShapeDtypeStruct((N, D), jnp.float32),
    )(nf, *flat_parts, W0n, W0a, b0, W1, b1, W2, b2, g, beta)


def kernel(node_features, edge_features, edge_index,
           edge_W0, edge_b0, edge_W1, edge_b1, edge_W2, edge_b2,
           edge_g, edge_beta,
           node_W0, node_b0, node_W1, node_b1, node_W2, node_b2,
           node_g, node_beta):
    N, DN = node_features.shape
    E, DE = edge_features.shape
    P = edge_W0.shape[0]
    n_chunks = E // CH
    cpp = n_chunks // PIECES
    sidx3d = edge_index[0].reshape(n_chunks, 1, CH)
    didx3d = edge_index[1].reshape(n_chunks, 1, CH)
    sidx_p = [sidx3d[k * cpp:(k + 1) * cpp] for k in range(PIECES)]
    didx_p = [didx3d[k * cpp:(k + 1) * cpp] for k in range(PIECES)]
    zeros = jnp.zeros((N, DE), dtype=jnp.float32)
    row = lambda b: b.reshape(1, -1)
    Ep = cpp * CH

    nf = node_features
    ef_p = [lax.slice_in_dim(edge_features, k * Ep, (k + 1) * Ep, axis=0)
            for k in range(PIECES)]
    for i in range(P):
        ew = (edge_W0[i, :DN], edge_W0[i, DN:2 * DN], edge_W0[i, 2 * DN:],
              row(edge_b0[i]), edge_W1[i], row(edge_b1[i]),
              edge_W2[i], row(edge_b2[i]), row(edge_g[i]), row(edge_beta[i]))
        parts_list = []
        new_ef_p = []
        for k in range(PIECES):
            src_rows, dst_rows = _sc_gather(nf, sidx_p[k], didx_p[k])
            efk = _tc_edge_mlp(src_rows, dst_rows, ef_p[k], *ew)
            new_ef_p.append(efk)
            parts_list.append(_sc_scatter(efk, didx_p[k], zeros, N))
        ef_p = new_ef_p
        nf = _tc_node_mlp(
            nf, parts_list,
            node_W0[i, :DN], node_W0[i, DN:],
            row(node_b0[i]), node_W1[i], row(node_b1[i]),
            node_W2[i], row(node_b2[i]), row(node_g[i]), row(node_beta[i]))
    return nf


# speedup vs baseline: 1.8392x; 1.0611x over previous
import functools

import jax
import jax.numpy as jnp
from jax import lax
from jax.experimental import pallas as pl
from jax.experimental.pallas import tpu as pltpu
from jax.experimental.pallas import tpu_sc as plsc

NC = 2
NS = 16
NW = NC * NS
CH = 128
PIECES = 2


def _sc_gather(table, sidx3d, didx3d):
    n_chunks = sidx3d.shape[0]
    _, D = table.shape
    Ep = n_chunks * CH
    mesh = plsc.VectorSubcoreMesh(core_axis_name="c", subcore_axis_name="s")

    @functools.partial(
        pl.kernel,
        out_type=(
            jax.ShapeDtypeStruct((Ep, D), jnp.float32),
            jax.ShapeDtypeStruct((Ep, D), jnp.float32),
        ),
        mesh=mesh,
        scratch_types=[
            pltpu.VMEM((1, CH), jnp.int32),
            pltpu.VMEM((1, CH), jnp.int32),
            pltpu.VMEM((1, CH), jnp.int32),
            pltpu.VMEM((1, CH), jnp.int32),
            pltpu.VMEM((CH, D), jnp.float32),
            pltpu.VMEM((CH, D), jnp.float32),
            pltpu.SemaphoreType.DMA,
            pltpu.SemaphoreType.DMA,
            pltpu.SemaphoreType.DMA,
            pltpu.SemaphoreType.DMA,
        ],
    )
    def k(table_hbm, sidx_hbm, didx_hbm, outs_hbm, outd_hbm,
          sidx_a, didx_a, sidx_b, didx_b, rows_a, rows_b,
          sem_g, sem_wa, sem_wb, sem_i):
        wid = lax.axis_index("s") * NC + lax.axis_index("c")
        nloc = (n_chunks - wid + NW - 1) // NW

        def chunk_of(j):
            return wid + j * NW

        def prefetch_async(j, si, di):
            @pl.when(j < nloc)
            def _():
                pltpu.async_copy(sidx_hbm.at[chunk_of(j)], si, sem_i)
                pltpu.async_copy(didx_hbm.at[chunk_of(j)], di, sem_i)

        def idx_wait(j, si, di):
            @pl.when(j > 0)
            def _():
                pltpu.make_async_copy(sidx_hbm.at[chunk_of(j)], si,
                                      sem_i).wait()
                pltpu.make_async_copy(didx_hbm.at[chunk_of(j)], di,
                                      sem_i).wait()

        def drain(rows, sem_w):
            pltpu.make_async_copy(table_hbm.at[pl.ds(0, CH)], rows,
                                  sem_w).wait()

        def unit(idx_v, rows, sem_w, out_hbm, c, do_drain):
            @pl.when(do_drain)
            def _():
                drain(rows, sem_w)

            pltpu.async_copy(table_hbm.at[idx_v.at[0]], rows, sem_g).wait()
            pltpu.async_copy(rows, out_hbm.at[pl.ds(c * CH, CH)], sem_w)

        @pl.when(nloc > 0)
        def _():
            pltpu.sync_copy(sidx_hbm.at[chunk_of(0)], sidx_a)
            pltpu.sync_copy(didx_hbm.at[chunk_of(0)], didx_a)

        @pl.loop(0, nloc)
        def _(j):
            c = chunk_of(j)
            even = j % 2 == 0

            @pl.when(even)
            def _():
                idx_wait(j, sidx_a, didx_a)
                prefetch_async(j + 1, sidx_b, didx_b)
                unit(sidx_a, rows_a, sem_wa, outs_hbm, c, j > 0)
                unit(didx_a, rows_b, sem_wb, outd_hbm, c, j > 0)

            @pl.when(jnp.logical_not(even))
            def _():
                idx_wait(j, sidx_b, didx_b)
                prefetch_async(j + 1, sidx_a, didx_a)
                unit(sidx_b, rows_a, sem_wa, outs_hbm, c, j > 0)
                unit(didx_b, rows_b, sem_wb, outd_hbm, c, j > 0)

        @pl.when(nloc > 0)
        def _():
            drain(rows_a, sem_wa)
            drain(rows_b, sem_wb)

    return k(table, sidx3d, didx3d)


def _sc_scatter(ef_piece, didx3d, zeros, n_nodes):
    n_chunks = didx3d.shape[0]
    D = ef_piece.shape[1]
    rpt = (n_nodes // NS) & ~7
    tail = n_nodes - rpt * NS
    mesh = plsc.VectorSubcoreMesh(core_axis_name="c", subcore_axis_name="s")

    @functools.partial(
        pl.kernel,
        out_type=jax.ShapeDtypeStruct((NC * n_nodes, D), jnp.float32),
        mesh=mesh,
        scratch_types=[
            pltpu.VMEM((1, CH), jnp.int32),
            pltpu.VMEM((1, CH), jnp.int32),
            pltpu.VMEM((CH, D), jnp.float32),
            pltpu.VMEM((CH, D), jnp.float32),
            pltpu.SemaphoreType.DMA,
            pltpu.SemaphoreType.DMA,
            pltpu.VMEM_SHARED((n_nodes, D), jnp.float32),
        ],
    )
    def k(ef_hbm, didx_hbm, zeros_hbm, out_hbm, idx_a, idx_b,
          rows_a, rows_b, sem_ra, sem_rb, acc_s):
        cid = lax.axis_index("c")
        sid = lax.axis_index("s")
        wid = sid * NC + cid
        pltpu.sync_copy(zeros_hbm.at[pl.ds(sid * rpt, rpt)],
                        acc_s.at[pl.ds(sid * rpt, rpt)])

        @pl.when(jnp.logical_and(sid == NS - 1, tail > 0))
        def _():
            pltpu.sync_copy(zeros_hbm.at[pl.ds(NS * rpt, tail)],
                            acc_s.at[pl.ds(NS * rpt, tail)])

        plsc.subcore_barrier()
        nloc = (n_chunks - wid + NW - 1) // NW

        def chunk_of(j):
            return wid + j * NW

        def prefetch(j, idx_v, rows, sem_r):
            @pl.when(j < nloc)
            def _():
                c = chunk_of(j)
                pltpu.sync_copy(didx_hbm.at[c], idx_v)
                pltpu.async_copy(ef_hbm.at[pl.ds(c * CH, CH)], rows, sem_r)

        def add_unit(idx_v, rows, sem_r):
            pltpu.make_async_copy(ef_hbm.at[pl.ds(0, CH)], rows,
                                  sem_r).wait()
            pltpu.sync_copy(rows, acc_s.at[idx_v.at[0]], add=True)

        prefetch(0, idx_a, rows_a, sem_ra)

        @pl.loop(0, nloc)
        def _(j):
            even = j % 2 == 0

            @pl.when(even)
            def _():
                prefetch(j + 1, idx_b, rows_b, sem_rb)
                add_unit(idx_a, rows_a, sem_ra)

            @pl.when(jnp.logical_not(even))
            def _():
                prefetch(j + 1, idx_a, rows_a, sem_ra)
                add_unit(idx_b, rows_b, sem_rb)

        plsc.subcore_barrier()
        pltpu.sync_copy(acc_s.at[pl.ds(sid * rpt, rpt)],
                        out_hbm.at[pl.ds(cid * n_nodes + sid * rpt, rpt)])

        @pl.when(jnp.logical_and(sid == NS - 1, tail > 0))
        def _():
            pltpu.sync_copy(acc_s.at[pl.ds(NS * rpt, tail)],
                            out_hbm.at[pl.ds(cid * n_nodes + NS * rpt, tail)])

    return k(ef_piece, didx3d, zeros)


def _layer_norm(h, g, beta):
    mu = jnp.mean(h, axis=-1, keepdims=True)
    var = jnp.mean((h - mu) * (h - mu), axis=-1, keepdims=True)
    return (h - mu) * lax.rsqrt(var + 1e-5) * g + beta


def _tc_edge_mlp(src, dst, ef, W0s, W0d, W0e, b0, W1, b1, W2, b2, g, beta):
    Ep, D = ef.shape
    BE = 2000
    grid = (Ep // BE,)

    def body(src_r, dst_r, ef_r, w0s_r, w0d_r, w0e_r, b0_r, w1_r, b1_r,
             w2_r, b2_r, g_r, beta_r, out_r):
        dot = functools.partial(jnp.dot, preferred_element_type=jnp.float32)
        x = (dot(src_r[...], w0s_r[...]) + dot(dst_r[...], w0d_r[...])
             + dot(ef_r[...], w0e_r[...]) + b0_r[...])
        h = jnp.maximum(x, 0.0)
        h = jnp.maximum(dot(h, w1_r[...]) + b1_r[...], 0.0)
        h = dot(h, w2_r[...]) + b2_r[...]
        out_r[...] = ef_r[...] + _layer_norm(h, g_r[...], beta_r[...])

    blk = lambda i: (i, 0)
    full = lambda i: (0, 0)
    return pl.pallas_call(
        body,
        grid=grid,
        in_specs=[
            pl.BlockSpec((BE, D), blk),
            pl.BlockSpec((BE, D), blk),
            pl.BlockSpec((BE, D), blk),
            pl.BlockSpec((D, D), full),
            pl.BlockSpec((D, D), full),
            pl.BlockSpec((D, D), full),
            pl.BlockSpec((1, D), full),
            pl.BlockSpec((D, D), full),
            pl.BlockSpec((1, D), full),
            pl.BlockSpec((D, D), full),
            pl.BlockSpec((1, D), full),
            pl.BlockSpec((1, D), full),
            pl.BlockSpec((1, D), full),
        ],
        out_specs=pl.BlockSpec((BE, D), blk),
        out_shape=jax.ShapeDtypeStruct((Ep, D), jnp.float32),
    )(src, dst, ef, W0s, W0d, W0e, b0, W1, b1, W2, b2, g, beta)


def _tc_node_mlp(nf, parts_list, W0n, W0a, b0, W1, b1, W2, b2, g, beta):
    N, D = nf.shape
    BN = 1000
    grid = (N // BN,)
    n_parts = 2 * len(parts_list)

    def body(*refs):
        nf_r = refs[0]
        part_rs = refs[1:1 + n_parts]
        (w0n_r, w0a_r, b0_r, w1_r, b1_r, w2_r, b2_r, g_r, beta_r,
         out_r) = refs[1 + n_parts:]
        dot = functools.partial(jnp.dot, preferred_element_type=jnp.float32)
        agg = part_rs[0][...]
        for pr in part_rs[1:]:
            agg = agg + pr[...]
        x = dot(nf_r[...], w0n_r[...]) + dot(agg, w0a_r[...]) + b0_r[...]
        h = jnp.maximum(x, 0.0)
        h = jnp.maximum(dot(h, w1_r[...]) + b1_r[...], 0.0)
        h = dot(h, w2_r[...]) + b2_r[...]
        out_r[...] = nf_r[...] + _layer_norm(h, g_r[...], beta_r[...])

    blk = lambda i: (i, 0)
    full = lambda i: (0, 0)
    flat_parts = []
    for parts in parts_list:
        flat_parts.append(lax.slice_in_dim(parts, 0, N, axis=0))
        flat_parts.append(lax.slice_in_dim(parts, N, 2 * N, axis=0))
    return pl.pallas_call(
        body,
        grid=grid,
        in_specs=(
            [pl.BlockSpec((BN, D), blk)] * (1 + n_parts)
            + [
                pl.BlockSpec((D, D), full),
                pl.BlockSpec((D, D), full),
                pl.BlockSpec((1, D), full),
                pl.BlockSpec((D, D), full),
                pl.BlockSpec((1, D), full),
                pl.BlockSpec((D, D), full),
                pl.BlockSpec((1, D), full),
                pl.BlockSpec((1, D), full),
                pl.BlockSpec((1, D), full),
            ]
        ),
        out_specs=pl.BlockSpec((BN, D), blk),
        out_shape=jax.ShapeDtypeStruct((N, D), jnp.float32),
    )(nf, *flat_parts, W0n, W0a, b0, W1, b1, W2, b2, g, beta)


def kernel(node_features, edge_features, edge_index,
           edge_W0, edge_b0, edge_W1, edge_b1, edge_W2, edge_b2,
           edge_g, edge_beta,
           node_W0, node_b0, node_W1, node_b1, node_W2, node_b2,
           node_g, node_beta):
    N, DN = node_features.shape
    E, DE = edge_features.shape
    P = edge_W0.shape[0]
    n_chunks = E // CH
    cpp = n_chunks // PIECES
    sidx3d = edge_index[0].reshape(n_chunks, 1, CH)
    didx3d = edge_index[1].reshape(n_chunks, 1, CH)
    sidx_p = [sidx3d[k * cpp:(k + 1) * cpp] for k in range(PIECES)]
    didx_p = [didx3d[k * cpp:(k + 1) * cpp] for k in range(PIECES)]
    zeros = jnp.zeros((N, DE), dtype=jnp.float32)
    row = lambda b: b.reshape(1, -1)
    Ep = cpp * CH

    nf = node_features
    ef_p = [lax.slice_in_dim(edge_features, k * Ep, (k + 1) * Ep, axis=0)
            for k in range(PIECES)]
    for i in range(P):
        ew = (edge_W0[i, :DN], edge_W0[i, DN:2 * DN], edge_W0[i, 2 * DN:],
              row(edge_b0[i]), edge_W1[i], row(edge_b1[i]),
              edge_W2[i], row(edge_b2[i]), row(edge_g[i]), row(edge_beta[i]))
        parts_list = []
        new_ef_p = []
        for k in range(PIECES):
            src_rows, dst_rows = _sc_gather(nf, sidx_p[k], didx_p[k])
            efk = _tc_edge_mlp(src_rows, dst_rows, ef_p[k], *ew)
            new_ef_p.append(efk)
            parts_list.append(_sc_scatter(efk, didx_p[k], zeros, N))
        ef_p = new_ef_p
        nf = _tc_node_mlp(
            nf, parts_list,
            node_W0[i, :DN], node_W0[i, DN:],
            row(node_b0[i]), node_W1[i], row(node_b1[i]),
            node_W2[i], row(node_b2[i]), row(node_g[i]), row(node_beta[i]))
    return nf


# speedup vs baseline: 1.8507x; 1.0062x over previous
import functools

import jax
import jax.numpy as jnp
from jax import lax
from jax.experimental import pallas as pl
from jax.experimental.pallas import tpu as pltpu
from jax.experimental.pallas import tpu_sc as plsc

NC = 2
NS = 16
NW = NC * NS
CH = 128
PIECES = 2


def _sc_gather(table, sidx3d, didx3d):
    n_chunks = sidx3d.shape[0]
    _, D = table.shape
    Ep = n_chunks * CH
    mesh = plsc.VectorSubcoreMesh(core_axis_name="c", subcore_axis_name="s")

    @functools.partial(
        pl.kernel,
        out_type=(
            jax.ShapeDtypeStruct((Ep, D), jnp.float32),
            jax.ShapeDtypeStruct((Ep, D), jnp.float32),
        ),
        mesh=mesh,
        scratch_types=[
            pltpu.VMEM((1, CH), jnp.int32),
            pltpu.VMEM((1, CH), jnp.int32),
            pltpu.VMEM((1, CH), jnp.int32),
            pltpu.VMEM((1, CH), jnp.int32),
            pltpu.VMEM((CH, D), jnp.float32),
            pltpu.VMEM((CH, D), jnp.float32),
            pltpu.SemaphoreType.DMA,
            pltpu.SemaphoreType.DMA,
            pltpu.SemaphoreType.DMA,
            pltpu.SemaphoreType.DMA,
        ],
    )
    def k(table_hbm, sidx_hbm, didx_hbm, outs_hbm, outd_hbm,
          sidx_a, didx_a, sidx_b, didx_b, rows_a, rows_b,
          sem_g, sem_wa, sem_wb, sem_i):
        wid = lax.axis_index("s") * NC + lax.axis_index("c")
        nloc = (n_chunks - wid + NW - 1) // NW

        def chunk_of(j):
            return wid + j * NW

        def prefetch_async(j, si, di):
            @pl.when(j < nloc)
            def _():
                pltpu.async_copy(sidx_hbm.at[chunk_of(j)], si, sem_i)
                pltpu.async_copy(didx_hbm.at[chunk_of(j)], di, sem_i)

        def idx_wait(j, si, di):
            @pl.when(j > 0)
            def _():
                pltpu.make_async_copy(sidx_hbm.at[chunk_of(j)], si,
                                      sem_i).wait()
                pltpu.make_async_copy(didx_hbm.at[chunk_of(j)], di,
                                      sem_i).wait()

        def drain(rows, sem_w):
            pltpu.make_async_copy(table_hbm.at[pl.ds(0, CH)], rows,
                                  sem_w).wait()

        def unit(idx_v, rows, sem_w, out_hbm, c, do_drain):
            @pl.when(do_drain)
            def _():
                drain(rows, sem_w)

            pltpu.async_copy(table_hbm.at[idx_v.at[0]], rows, sem_g).wait()
            pltpu.async_copy(rows, out_hbm.at[pl.ds(c * CH, CH)], sem_w)

        @pl.when(nloc > 0)
        def _():
            pltpu.sync_copy(sidx_hbm.at[chunk_of(0)], sidx_a)
            pltpu.sync_copy(didx_hbm.at[chunk_of(0)], didx_a)

        @pl.loop(0, nloc)
        def _(j):
            c = chunk_of(j)
            even = j % 2 == 0

            @pl.when(even)
            def _():
                idx_wait(j, sidx_a, didx_a)
                prefetch_async(j + 1, sidx_b, didx_b)
                unit(sidx_a, rows_a, sem_wa, outs_hbm, c, j > 0)
                unit(didx_a, rows_b, sem_wb, outd_hbm, c, j > 0)

            @pl.when(jnp.logical_not(even))
            def _():
                idx_wait(j, sidx_b, didx_b)
                prefetch_async(j + 1, sidx_a, didx_a)
                unit(sidx_b, rows_a, sem_wa, outs_hbm, c, j > 0)
                unit(didx_b, rows_b, sem_wb, outd_hbm, c, j > 0)

        @pl.when(nloc > 0)
        def _():
            drain(rows_a, sem_wa)
            drain(rows_b, sem_wb)

    return k(table, sidx3d, didx3d)


def _sc_scatter(ef_piece, didx3d, zeros, n_nodes):
    n_chunks = didx3d.shape[0]
    D = ef_piece.shape[1]
    rpt = (n_nodes // NS) & ~7
    tail = n_nodes - rpt * NS
    mesh = plsc.VectorSubcoreMesh(core_axis_name="c", subcore_axis_name="s")

    @functools.partial(
        pl.kernel,
        out_type=jax.ShapeDtypeStruct((NC * n_nodes, D), jnp.float32),
        mesh=mesh,
        scratch_types=[
            pltpu.VMEM((1, CH), jnp.int32),
            pltpu.VMEM((1, CH), jnp.int32),
            pltpu.VMEM((CH, D), jnp.float32),
            pltpu.VMEM((CH, D), jnp.float32),
            pltpu.SemaphoreType.DMA,
            pltpu.SemaphoreType.DMA,
            pltpu.SemaphoreType.DMA,
            pltpu.VMEM_SHARED((n_nodes, D), jnp.float32),
        ],
    )
    def k(ef_hbm, didx_hbm, zeros_hbm, out_hbm, idx_a, idx_b,
          rows_a, rows_b, sem_ra, sem_rb, sem_i, acc_s):
        cid = lax.axis_index("c")
        sid = lax.axis_index("s")
        wid = sid * NC + cid
        pltpu.sync_copy(zeros_hbm.at[pl.ds(sid * rpt, rpt)],
                        acc_s.at[pl.ds(sid * rpt, rpt)])

        @pl.when(jnp.logical_and(sid == NS - 1, tail > 0))
        def _():
            pltpu.sync_copy(zeros_hbm.at[pl.ds(NS * rpt, tail)],
                            acc_s.at[pl.ds(NS * rpt, tail)])

        plsc.subcore_barrier()
        nloc = (n_chunks - wid + NW - 1) // NW

        def chunk_of(j):
            return wid + j * NW

        def prefetch(j, idx_v, rows, sem_r, sync_idx):
            @pl.when(j < nloc)
            def _():
                c = chunk_of(j)
                if sync_idx:
                    pltpu.sync_copy(didx_hbm.at[c], idx_v)
                else:
                    pltpu.async_copy(didx_hbm.at[c], idx_v, sem_i)
                pltpu.async_copy(ef_hbm.at[pl.ds(c * CH, CH)], rows, sem_r)

        def add_unit(j, idx_v, rows, sem_r):
            @pl.when(j > 0)
            def _():
                pltpu.make_async_copy(didx_hbm.at[chunk_of(j)], idx_v,
                                      sem_i).wait()

            pltpu.make_async_copy(ef_hbm.at[pl.ds(0, CH)], rows,
                                  sem_r).wait()
            pltpu.sync_copy(rows, acc_s.at[idx_v.at[0]], add=True)

        prefetch(0, idx_a, rows_a, sem_ra, True)

        @pl.loop(0, nloc)
        def _(j):
            even = j % 2 == 0

            @pl.when(even)
            def _():
                prefetch(j + 1, idx_b, rows_b, sem_rb, False)
                add_unit(j, idx_a, rows_a, sem_ra)

            @pl.when(jnp.logical_not(even))
            def _():
                prefetch(j + 1, idx_a, rows_a, sem_ra, False)
                add_unit(j, idx_b, rows_b, sem_rb)

        plsc.subcore_barrier()
        pltpu.sync_copy(acc_s.at[pl.ds(sid * rpt, rpt)],
                        out_hbm.at[pl.ds(cid * n_nodes + sid * rpt, rpt)])

        @pl.when(jnp.logical_and(sid == NS - 1, tail > 0))
        def _():
            pltpu.sync_copy(acc_s.at[pl.ds(NS * rpt, tail)],
                            out_hbm.at[pl.ds(cid * n_nodes + NS * rpt, tail)])

    return k(ef_piece, didx3d, zeros)


def _layer_norm(h, g, beta):
    mu = jnp.mean(h, axis=-1, keepdims=True)
    var = jnp.mean((h - mu) * (h - mu), axis=-1, keepdims=True)
    return (h - mu) * lax.rsqrt(var + 1e-5) * g + beta


def _tc_edge_mlp(src, dst, ef, W0s, W0d, W0e, b0, W1, b1, W2, b2, g, beta):
    Ep, D = ef.shape
    BE = 2000
    grid = (Ep // BE,)

    def body(src_r, dst_r, ef_r, w0s_r, w0d_r, w0e_r, b0_r, w1_r, b1_r,
             w2_r, b2_r, g_r, beta_r, out_r):
        dot = functools.partial(jnp.dot, preferred_element_type=jnp.float32)
        x = (dot(src_r[...], w0s_r[...]) + dot(dst_r[...], w0d_r[...])
             + dot(ef_r[...], w0e_r[...]) + b0_r[...])
        h = jnp.maximum(x, 0.0)
        h = jnp.maximum(dot(h, w1_r[...]) + b1_r[...], 0.0)
        h = dot(h, w2_r[...]) + b2_r[...]
        out_r[...] = ef_r[...] + _layer_norm(h, g_r[...], beta_r[...])

    blk = lambda i: (i, 0)
    full = lambda i: (0, 0)
    return pl.pallas_call(
        body,
        grid=grid,
        in_specs=[
            pl.BlockSpec((BE, D), blk),
            pl.BlockSpec((BE, D), blk),
            pl.BlockSpec((BE, D), blk),
            pl.BlockSpec((D, D), full),
            pl.BlockSpec((D, D), full),
            pl.BlockSpec((D, D), full),
            pl.BlockSpec((1, D), full),
            pl.BlockSpec((D, D), full),
            pl.BlockSpec((1, D), full),
            pl.BlockSpec((D, D), full),
            pl.BlockSpec((1, D), full),
            pl.BlockSpec((1, D), full),
            pl.BlockSpec((1, D), full),
        ],
        out_specs=pl.BlockSpec((BE, D), blk),
        out_shape=jax.ShapeDtypeStruct((Ep, D), jnp.float32),
    )(src, dst, ef, W0s, W0d, W0e, b0, W1, b1, W2, b2, g, beta)


def _tc_node_mlp(nf, parts_list, W0n, W0a, b0, W1, b1, W2, b2, g, beta):
    N, D = nf.shape
    BN = 1000
    grid = (N // BN,)
    n_parts = 2 * len(parts_list)

    def body(*refs):
        nf_r = refs[0]
        part_rs = refs[1:1 + n_parts]
        (w0n_r, w0a_r, b0_r, w1_r, b1_r, w2_r, b2_r, g_r, beta_r,
         out_r) = refs[1 + n_parts:]
        dot = functools.partial(jnp.dot, preferred_element_type=jnp.float32)
        agg = part_rs[0][...]
        for pr in part_rs[1:]:
            agg = agg + pr[...]
        x = dot(nf_r[...], w0n_r[...]) + dot(agg, w0a_r[...]) + b0_r[...]
        h = jnp.maximum(x, 0.0)
        h = jnp.maximum(dot(h, w1_r[...]) + b1_r[...], 0.0)
        h = dot(h, w2_r[...]) + b2_r[...]
        out_r[...] = nf_r[...] + _layer_norm(h, g_r[...], beta_r[...])

    blk = lambda i: (i, 0)
    full = lambda i: (0, 0)
    flat_parts = []
    for parts in parts_list:
        flat_parts.append(lax.slice_in_dim(parts, 0, N, axis=0))
        flat_parts.append(lax.slice_in_dim(parts, N, 2 * N, axis=0))
    return pl.pallas_call(
        body,
        grid=grid,
        in_specs=(
            [pl.BlockSpec((BN, D), blk)] * (1 + n_parts)
            + [
                pl.BlockSpec((D, D), full),
                pl.BlockSpec((D, D), full),
                pl.BlockSpec((1, D), full),
                pl.BlockSpec((D, D), full),
                pl.BlockSpec((1, D), full),
                pl.BlockSpec((D, D), full),
                pl.BlockSpec((1, D), full),
                pl.BlockSpec((1, D), full),
                pl.BlockSpec((1, D), full),
            ]
        ),
        out_specs=pl.BlockSpec((BN, D), blk),
        out_shape=jax.ShapeDtypeStruct((N, D), jnp.float32),
    )(nf, *flat_parts, W0n, W0a, b0, W1, b1, W2, b2, g, beta)


def kernel(node_features, edge_features, edge_index,
           edge_W0, edge_b0, edge_W1, edge_b1, edge_W2, edge_b2,
           edge_g, edge_beta,
           node_W0, node_b0, node_W1, node_b1, node_W2, node_b2,
           node_g, node_beta):
    N, DN = node_features.shape
    E, DE = edge_features.shape
    P = edge_W0.shape[0]
    n_chunks = E // CH
    cpp = n_chunks // PIECES
    sidx3d = edge_index[0].reshape(n_chunks, 1, CH)
    didx3d = edge_index[1].reshape(n_chunks, 1, CH)
    sidx_p = [sidx3d[k * cpp:(k + 1) * cpp] for k in range(PIECES)]
    didx_p = [didx3d[k * cpp:(k + 1) * cpp] for k in range(PIECES)]
    zeros = jnp.zeros((N, DE), dtype=jnp.float32)
    row = lambda b: b.reshape(1, -1)
    Ep = cpp * CH

    nf = node_features
    ef_p = [lax.slice_in_dim(edge_features, k * Ep, (k + 1) * Ep, axis=0)
            for k in range(PIECES)]
    for i in range(P):
        ew = (edge_W0[i, :DN], edge_W0[i, DN:2 * DN], edge_W0[i, 2 * DN:],
              row(edge_b0[i]), edge_W1[i], row(edge_b1[i]),
              edge_W2[i], row(edge_b2[i]), row(edge_g[i]), row(edge_beta[i]))
        parts_list = []
        new_ef_p = []
        for k in range(PIECES):
            src_rows, dst_rows = _sc_gather(nf, sidx_p[k], didx_p[k])
            efk = _tc_edge_mlp(src_rows, dst_rows, ef_p[k], *ew)
            new_ef_p.append(efk)
            parts_list.append(_sc_scatter(efk, didx_p[k], zeros, N))
        ef_p = new_ef_p
        nf = _tc_node_mlp(
            nf, parts_list,
            node_W0[i, :DN], node_W0[i, DN:],
            row(node_b0[i]), node_W1[i], row(node_b1[i]),
            node_W2[i], row(node_b2[i]), row(node_g[i]), row(node_beta[i]))
    return nf
